# Initial kernel scaffold; baseline (speedup 1.0000x reference)
import jax
import jax.numpy as jnp
from jax.experimental import pallas as pl


def kernel(user, user_items, prev_bundles, prev_bundle_items, prev_user_bundle_overlap_items, embedding_table, W0, b0, W1, b1, W2, b2, W3, b3, W4, b4):
    raise NotImplementedError("write your pallas kernel here")


# speedup vs baseline: 1.9484x; 1.9484x over previous
import functools

import jax
import jax.numpy as jnp
import numpy as np
from jax import lax
from jax.experimental import pallas as pl
from jax.experimental.pallas import tpu as pltpu
from jax.experimental.pallas import tpu_sc as plsc

USER_NUM = 100000
BUNDLE_NUM = 1000
ITEM_NUM = 100000
TOTAL = USER_NUM + BUNDLE_NUM + ITEM_NUM
EMBED_DIM = 128
B = 1024
NGROUPS = 5
ELEMS = 880
NW = 32
ROWS_PER_W = B // NW
CHUNK = 128
CHUNKS_PER_W = ROWS_PER_W * ELEMS // CHUNK
ACC_ROWS = 16 * ROWS_PER_W * NGROUPS
SHARED_ROWS = ACC_ROWS + 16 * 8


def _sc_pool_body(table_hbm, idx_hbm, out_hbm,
                  idx_v, buf0, buf1, dst0, dst1, shared,
                  gs0, gs1, ss0, ss1):
    c = lax.axis_index("c")
    s = lax.axis_index("s")
    w = c * 16 + s

    pltpu.sync_copy(idx_hbm.at[w], idx_v)

    zv = jnp.zeros((16,), jnp.float32)

    def _zrow(i, carry):
        for v in range(8):
            buf0[i, pl.ds(v * 16, 16)] = zv
        return carry

    lax.fori_loop(0, CHUNK, _zrow, 0)
    acc0 = s * (ROWS_PER_W * NGROUPS)
    pltpu.sync_copy(buf0, shared.at[pl.ds(acc0, 128)])
    pltpu.sync_copy(buf0.at[pl.ds(0, 32)], shared.at[pl.ds(acc0 + 128, 32)])
    pltpu.sync_copy(buf0.at[pl.ds(0, 8)], shared.at[pl.ds(ACC_ROWS + 8 * s, 8)])

    i32 = jnp.int32
    trash16 = jnp.broadcast_to(ACC_ROWS + 8 * s, (16,)).astype(i32)
    boff = s * (ROWS_PER_W * NGROUPS)
    iota16 = lax.iota(i32, 16)
    one16 = jnp.full((16,), 1, i32)
    zero16 = jnp.full((16,), 0, i32)

    def _c16(v):
        return jnp.full((16,), v, i32)

    def compute_dst(j, dref):
        base0 = j * CHUNK
        for v in range(8):
            base = base0 + v * 16
            r = base // ELEMS
            p = base % ELEMS
            pv = jnp.broadcast_to(p, (16,)).astype(i32) + iota16
            g = zero16
            for bnd in (1, 51, 71, 471, 871):
                g = g + jnp.where(pv >= _c16(bnd), one16, zero16)
            sent = jnp.where(g == _c16(2), _c16(USER_NUM + BUNDLE_NUM),
                             jnp.where(g == zero16, _c16(-1), _c16(TOTAL)))
            iv = idx_v[j, pl.ds(v * 16, 16)]
            rowbase = jnp.broadcast_to(boff + r * NGROUPS, (16,)).astype(i32)
            dref[pl.ds(v * 16, 16)] = jnp.where(
                iv == sent, trash16, rowbase + g)

    def gissue(j, buf, sem):
        pltpu.async_copy(table_hbm.at[idx_v.at[j]], buf, sem)

    def gwait(buf, sem):
        pltpu.make_async_copy(table_hbm.at[idx_v.at[0]], buf, sem).wait()

    def sissue(buf, dref, sem):
        pltpu.async_copy(buf, shared.at[dref], sem, add=True)

    def swait(buf, dref, sem):
        pltpu.make_async_copy(buf, shared.at[dref], sem).wait()

    gissue(0, buf0, gs0)

    def pair(j2, carry):
        j0 = j2 * 2
        gwait(buf0, gs0)
        compute_dst(j0, dst0)

        @pl.when(j2 > 0)
        def _():
            swait(buf1, dst1, ss1)

        gissue(j0 + 1, buf1, gs1)
        sissue(buf0, dst0, ss0)
        gwait(buf1, gs1)
        compute_dst(j0 + 1, dst1)

        @pl.when(j2 < CHUNKS_PER_W // 2 - 1)
        def _():
            swait(buf0, dst0, ss0)
            gissue(j0 + 2, buf0, gs0)

        sissue(buf1, dst1, ss1)
        return carry

    lax.fori_loop(0, CHUNKS_PER_W // 2, pair, 0)
    swait(buf0, dst0, ss0)
    swait(buf1, dst1, ss1)

    nout = ROWS_PER_W * NGROUPS
    pltpu.sync_copy(shared.at[pl.ds(acc0, nout)],
                    out_hbm.at[pl.ds(c * ACC_ROWS + s * nout, nout)])


def _sc_pool(table, idx_flat):
    mesh = plsc.VectorSubcoreMesh(core_axis_name="c", subcore_axis_name="s")
    f = functools.partial(
        pl.kernel,
        out_type=jax.ShapeDtypeStruct((B * NGROUPS, EMBED_DIM), jnp.float32),
        mesh=mesh,
        scratch_types=[
            pltpu.VMEM((CHUNKS_PER_W, CHUNK), jnp.int32),
            pltpu.VMEM((CHUNK, EMBED_DIM), jnp.float32),
            pltpu.VMEM((CHUNK, EMBED_DIM), jnp.float32),
            pltpu.VMEM((CHUNK,), jnp.int32),
            pltpu.VMEM((CHUNK,), jnp.int32),
            pltpu.VMEM_SHARED((SHARED_ROWS, EMBED_DIM), jnp.float32),
            pltpu.SemaphoreType.DMA,
            pltpu.SemaphoreType.DMA,
            pltpu.SemaphoreType.DMA,
            pltpu.SemaphoreType.DMA,
        ],
    )(_sc_pool_body)
    return f(table, idx_flat)


def _mlp_body(x_ref, w0, b0, w1, b1, w2, b2, w3, b3, w4, b4, o_ref):
    h = x_ref[...]
    for wr, br, last in ((w0, b0, False), (w1, b1, False), (w2, b2, False),
                         (w3, b3, False), (w4, b4, True)):
        h = jnp.dot(h, wr[...], preferred_element_type=jnp.float32) + br[...]
        if not last:
            h = jnp.maximum(h, 0.0)
    o_ref[...] = h


def _mlp(x, Ws, bs):
    bm = 256
    grid = (B // bm,)
    hid = Ws[0].shape[0]
    in_specs = [pl.BlockSpec((bm, hid), lambda i: (i, 0))]
    for wmat in Ws:
        d_in, d_out = wmat.shape
        in_specs.append(pl.BlockSpec((d_in, d_out), lambda i: (0, 0)))
        in_specs.append(pl.BlockSpec((1, d_out), lambda i: (0, 0)))
    args = [x]
    for wmat, bvec in zip(Ws, bs):
        args.append(wmat)
        args.append(bvec.reshape(1, -1))
    return pl.pallas_call(
        _mlp_body,
        grid=grid,
        in_specs=in_specs,
        out_specs=pl.BlockSpec((bm, EMBED_DIM), lambda i: (i, 0)),
        out_shape=jax.ShapeDtypeStruct((B, EMBED_DIM), jnp.float32),
    )(*args)


def kernel(user, user_items, prev_bundles, prev_bundle_items,
           prev_user_bundle_overlap_items, embedding_table,
           W0, b0, W1, b1, W2, b2, W3, b3, W4, b4):
    i32 = jnp.int32
    idx = jnp.concatenate([
        user.reshape(B, 1).astype(i32),
        user_items.reshape(B, 50).astype(i32),
        prev_bundles.reshape(B, 20).astype(i32),
        prev_bundle_items.reshape(B, 400).astype(i32),
        prev_user_bundle_overlap_items.reshape(B, 400).astype(i32),
        jnp.full((B, ELEMS - 871), TOTAL, dtype=i32),
    ], axis=1)
    idx_flat = idx.reshape(NW, CHUNKS_PER_W, CHUNK)
    table = embedding_table.astype(jnp.float32)
    x5 = _sc_pool(table, idx_flat)
    x = x5.reshape(B, NGROUPS * EMBED_DIM)
    out = _mlp(x, [W0, W1, W2, W3, W4], [b0, b1, b2, b3, b4])
    return out


# speedup vs baseline: 1.9490x; 1.0003x over previous
import functools

import jax
import jax.numpy as jnp
import numpy as np
from jax import lax
from jax.experimental import pallas as pl
from jax.experimental.pallas import tpu as pltpu
from jax.experimental.pallas import tpu_sc as plsc

USER_NUM = 100000
BUNDLE_NUM = 1000
ITEM_NUM = 100000
TOTAL = USER_NUM + BUNDLE_NUM + ITEM_NUM
EMBED_DIM = 128
B = 1024
NGROUPS = 5
ELEMS = 880
NW = 32
ROWS_PER_W = B // NW
CHUNK = 128
CHUNKS_PER_W = ROWS_PER_W * ELEMS // CHUNK
ACC_ROWS = 16 * ROWS_PER_W * NGROUPS
SHARED_ROWS = ACC_ROWS + 16 * 8


def _sc_pool_body(table_hbm, idx_hbm, out_hbm,
                  idx_v, buf0, buf1, buf2, buf3, dst0, dst1, dst2, dst3,
                  shared, gs0, gs1, gs2, gs3, ss0, ss1, ss2, ss3):
    c = lax.axis_index("c")
    s = lax.axis_index("s")
    w = c * 16 + s

    pltpu.sync_copy(idx_hbm.at[w], idx_v)

    zv = jnp.zeros((16,), jnp.float32)

    def _zrow(i, carry):
        for v in range(8):
            buf0[i, pl.ds(v * 16, 16)] = zv
        return carry

    lax.fori_loop(0, CHUNK, _zrow, 0)
    acc0 = s * (ROWS_PER_W * NGROUPS)
    pltpu.sync_copy(buf0, shared.at[pl.ds(acc0, 128)])
    pltpu.sync_copy(buf0.at[pl.ds(0, 32)], shared.at[pl.ds(acc0 + 128, 32)])
    pltpu.sync_copy(buf0.at[pl.ds(0, 8)], shared.at[pl.ds(ACC_ROWS + 8 * s, 8)])

    i32 = jnp.int32
    trash16 = jnp.broadcast_to(ACC_ROWS + 8 * s, (16,)).astype(i32)
    boff = s * (ROWS_PER_W * NGROUPS)
    iota16 = lax.iota(i32, 16)
    one16 = jnp.full((16,), 1, i32)
    zero16 = jnp.full((16,), 0, i32)

    def _c16(v):
        return jnp.full((16,), v, i32)

    def compute_dst(j, dref):
        base0 = j * CHUNK
        for v in range(8):
            base = base0 + v * 16
            r = base // ELEMS
            p = base % ELEMS
            pv = jnp.broadcast_to(p, (16,)).astype(i32) + iota16
            g = zero16
            for bnd in (1, 51, 71, 471, 871):
                g = g + jnp.where(pv >= _c16(bnd), one16, zero16)
            sent = jnp.where(g == _c16(2), _c16(USER_NUM + BUNDLE_NUM),
                             jnp.where(g == zero16, _c16(-1), _c16(TOTAL)))
            iv = idx_v[j, pl.ds(v * 16, 16)]
            rowbase = jnp.broadcast_to(boff + r * NGROUPS, (16,)).astype(i32)
            dref[pl.ds(v * 16, 16)] = jnp.where(
                iv == sent, trash16, rowbase + g)

    def gissue(j, buf, sem):
        pltpu.async_copy(table_hbm.at[idx_v.at[j]], buf, sem)

    def gwait(buf, sem):
        pltpu.make_async_copy(table_hbm.at[idx_v.at[0]], buf, sem).wait()

    def sissue(buf, dref, sem):
        pltpu.async_copy(buf, shared.at[dref], sem, add=True)

    def swait(buf, dref, sem):
        pltpu.make_async_copy(buf, shared.at[dref], sem).wait()

    bufs = (buf0, buf1, buf2, buf3)
    dsts = (dst0, dst1, dst2, dst3)
    gss = (gs0, gs1, gs2, gs3)
    sss = (ss0, ss1, ss2, ss3)
    NB = 4
    NGRP = CHUNKS_PER_W // NB

    for t in range(NB - 1):
        gissue(t, bufs[t], gss[t])

    def quad(j2, carry):
        for t in range(NB):
            j = j2 * NB + t
            compute_dst(j, dsts[t])
            gwait(bufs[t], gss[t])
            sissue(bufs[t], dsts[t], sss[t])
            tn = (t + NB - 1) % NB
            jn = j + NB - 1
            if t == 0:
                @pl.when(j2 > 0)
                def _():
                    swait(bufs[tn], dsts[tn], sss[tn])
                gissue(jn, bufs[tn], gss[tn])
            else:
                @pl.when(j2 < NGRP - 1)
                def _():
                    swait(bufs[tn], dsts[tn], sss[tn])
                    gissue(jn, bufs[tn], gss[tn])
        return carry

    lax.fori_loop(0, NGRP, quad, 0)
    for t in range(NB):
        swait(bufs[t], dsts[t], sss[t])

    nout = ROWS_PER_W * NGROUPS
    pltpu.sync_copy(shared.at[pl.ds(acc0, nout)],
                    out_hbm.at[pl.ds(c * ACC_ROWS + s * nout, nout)])


def _sc_pool(table, idx_flat):
    mesh = plsc.VectorSubcoreMesh(core_axis_name="c", subcore_axis_name="s")
    f = functools.partial(
        pl.kernel,
        out_type=jax.ShapeDtypeStruct((B * NGROUPS, EMBED_DIM), jnp.float32),
        mesh=mesh,
        scratch_types=[
            pltpu.VMEM((CHUNKS_PER_W, CHUNK), jnp.int32),
            pltpu.VMEM((CHUNK, EMBED_DIM), jnp.float32),
            pltpu.VMEM((CHUNK, EMBED_DIM), jnp.float32),
            pltpu.VMEM((CHUNK, EMBED_DIM), jnp.float32),
            pltpu.VMEM((CHUNK, EMBED_DIM), jnp.float32),
            pltpu.VMEM((CHUNK,), jnp.int32),
            pltpu.VMEM((CHUNK,), jnp.int32),
            pltpu.VMEM((CHUNK,), jnp.int32),
            pltpu.VMEM((CHUNK,), jnp.int32),
            pltpu.VMEM_SHARED((SHARED_ROWS, EMBED_DIM), jnp.float32),
        ] + [pltpu.SemaphoreType.DMA] * 8,
    )(_sc_pool_body)
    return f(table, idx_flat)


def _mlp_body(x_ref, w0, b0, w1, b1, w2, b2, w3, b3, w4, b4, o_ref):
    h = x_ref[...]
    for wr, br, last in ((w0, b0, False), (w1, b1, False), (w2, b2, False),
                         (w3, b3, False), (w4, b4, True)):
        h = jnp.dot(h, wr[...], preferred_element_type=jnp.float32) + br[...]
        if not last:
            h = jnp.maximum(h, 0.0)
    o_ref[...] = h


def _mlp(x, Ws, bs):
    bm = 256
    grid = (B // bm,)
    hid = Ws[0].shape[0]
    in_specs = [pl.BlockSpec((bm, hid), lambda i: (i, 0))]
    for wmat in Ws:
        d_in, d_out = wmat.shape
        in_specs.append(pl.BlockSpec((d_in, d_out), lambda i: (0, 0)))
        in_specs.append(pl.BlockSpec((1, d_out), lambda i: (0, 0)))
    args = [x]
    for wmat, bvec in zip(Ws, bs):
        args.append(wmat)
        args.append(bvec.reshape(1, -1))
    return pl.pallas_call(
        _mlp_body,
        grid=grid,
        in_specs=in_specs,
        out_specs=pl.BlockSpec((bm, EMBED_DIM), lambda i: (i, 0)),
        out_shape=jax.ShapeDtypeStruct((B, EMBED_DIM), jnp.float32),
    )(*args)


def kernel(user, user_items, prev_bundles, prev_bundle_items,
           prev_user_bundle_overlap_items, embedding_table,
           W0, b0, W1, b1, W2, b2, W3, b3, W4, b4):
    i32 = jnp.int32
    idx = jnp.concatenate([
        user.reshape(B, 1).astype(i32),
        user_items.reshape(B, 50).astype(i32),
        prev_bundles.reshape(B, 20).astype(i32),
        prev_bundle_items.reshape(B, 400).astype(i32),
        prev_user_bundle_overlap_items.reshape(B, 400).astype(i32),
        jnp.full((B, ELEMS - 871), TOTAL, dtype=i32),
    ], axis=1)
    idx_flat = idx.reshape(NW, CHUNKS_PER_W, CHUNK)
    table = embedding_table.astype(jnp.float32)
    x5 = _sc_pool(table, idx_flat)
    x = x5.reshape(B, NGROUPS * EMBED_DIM)
    out = _mlp(x, [W0, W1, W2, W3, W4], [b0, b1, b2, b3, b4])
    return out
